# baseline (speedup 1.0000x reference)
import functools

import jax
import jax.numpy as jnp
from jax import lax
from jax.experimental import pallas as pl
from jax.experimental.pallas import tpu as pltpu
from jax.experimental.pallas import tpu_sc as plsc

_B = 16384
_D = 256
_L = 16

_NC = 2
_NS = 16
_NW = _NC * _NS
_ROWS_PER_W = _B // _NW

_R = 64
_T = _ROWS_PER_W // _R


def _sc_body(pe_hbm, out_hbm, pe_v, buf, sem):
    wid = lax.axis_index("s") * _NC + lax.axis_index("c")
    base = wid * _ROWS_PER_W

    pltpu.sync_copy(pe_hbm, pe_v)
    vecs = [pe_v[0, pl.ds(j * _L, _L)] for j in range(_D // _L)]

    def fill_row(r, carry):
        for j in range(_D // _L):
            buf[r, pl.ds(j * _L, _L)] = vecs[j]
        return carry

    lax.fori_loop(0, _R, fill_row, 0)

    copies = [
        pltpu.async_copy(
            buf, out_hbm.at[pl.ds(base + t * _R, _R), 0], sem
        )
        for t in range(_T)
    ]
    for c in copies:
        c.wait()


def kernel(batch_size, pos_embed):
    mesh = plsc.VectorSubcoreMesh(core_axis_name="c", subcore_axis_name="s")
    return pl.kernel(
        _sc_body,
        out_type=jax.ShapeDtypeStruct((_B, 1, _D), jnp.float32),
        mesh=mesh,
        scratch_types=[
            pltpu.VMEM((1, _D), jnp.float32),
            pltpu.VMEM((_R, _D), jnp.float32),
            pltpu.SemaphoreType.DMA,
        ],
    )(pos_embed)


# speedup vs baseline: 3.3212x; 3.3212x over previous
import jax
import jax.numpy as jnp
from jax.experimental import pallas as pl

_B = 16384
_D = 256
_BLOCK = 4096


def _body(pe_ref, out_ref):
    out_ref[...] = jnp.broadcast_to(pe_ref[...][None], out_ref.shape)


def kernel(batch_size, pos_embed):
    out = pl.pallas_call(
        _body,
        grid=(_B // _BLOCK,),
        in_specs=[pl.BlockSpec((1, _D), lambda i: (0, 0))],
        out_specs=pl.BlockSpec((_BLOCK, 1, _D), lambda i: (i, 0, 0)),
        out_shape=jax.ShapeDtypeStruct((_B, 1, _D), jnp.float32),
    )(pos_embed)
    return out
